# Initial kernel scaffold; baseline (speedup 1.0000x reference)
import jax
import jax.numpy as jnp
from jax.experimental import pallas as pl


def kernel(x, edge_index, init_w1, w1, root_w1, b1, init_w2, w2, root_w2, b2):
    raise NotImplementedError("write your pallas kernel here")


# speedup vs baseline: 27.2319x; 27.2319x over previous
import functools

import jax
import jax.numpy as jnp
from jax import lax
from jax.experimental import pallas as pl
from jax.experimental.pallas import tpu as pltpu
from jax.experimental.pallas import tpu_sc as plsc

N = 10000
E = 160000
F_IN = 256
HID = 256
NCLS = 64
K = 2

N_PAD = 10240
NT = 16
NSC = 2
RPT = N_PAD // NT
E_PAD = 163840
B_E = 128
NB_E = E_PAD // NT // B_E
B_H = 128
NB_H = E_PAD // (NSC * NT) // B_H

KU = 8
RB = 400
GR = N // RB


def _sc_mesh():
  return plsc.VectorSubcoreMesh(core_axis_name="c", subcore_axis_name="s")


def _zero_rows(rows, nrows, fc):
  z16 = jnp.zeros((16,), jnp.float32)

  def zrow(i, carry):
    def zcol(jj, carry2):
      rows[i, pl.ds(jj * 16, 16)] = z16
      return carry2
    return lax.fori_loop(0, fc // 16, zcol, carry)
  lax.fori_loop(0, nrows, zrow, 0)


def _make_spmm_fsplit(ch, cpc, fc):
  @functools.partial(
      pl.kernel,
      out_type=jax.ShapeDtypeStruct((ch * N_PAD, fc), jnp.float32),
      mesh=_sc_mesh(),
      scratch_types=[
          pltpu.VMEM((KU, B_E), jnp.int32),
          pltpu.VMEM((KU, B_E), jnp.int32),
          pltpu.VMEM((B_E, fc), jnp.float32),
          pltpu.VMEM_SHARED((N_PAD, fc), jnp.float32),
          pltpu.SemaphoreType.DMA,
      ],
  )
  def spmm(h_hbm, src_hbm, dst_hbm, out_hbm, sstage, dstage, rows, acc, sem):
    c = lax.axis_index("c")
    s = lax.axis_index("s")
    r0 = s * RPT

    for j in range(cpc):
      cg = c * cpc + j
      base = cg * N

      _zero_rows(rows, 40, fc)
      for b in range(RPT // 40):
        pltpu.sync_copy(rows.at[pl.ds(0, 40)],
                        acc.at[pl.ds(r0 + b * 40, 40)])
      plsc.subcore_barrier()

      def rnd(g, carry):
        pltpu.sync_copy(src_hbm.at[s].at[pl.ds(g * KU, KU)], sstage)
        pltpu.sync_copy(dst_hbm.at[s].at[pl.ds(g * KU, KU)], dstage)
        for jj in range(KU):
          for v in range(B_E // 16):
            sl = pl.ds(v * 16, 16)
            sstage[jj, sl] = sstage[jj, sl] + base
        for jj in range(KU):
          pltpu.async_copy(h_hbm.at[sstage.at[jj]], rows, sem).wait()
          pltpu.sync_copy(rows, acc.at[dstage.at[jj]], add=True)
        return carry
      lax.fori_loop(0, NB_E // KU, rnd, 0)
      plsc.subcore_barrier()
      pltpu.sync_copy(acc.at[pl.ds(r0, RPT)],
                      out_hbm.at[pl.ds(cg * N_PAD + r0, RPT)])
    return

  return spmm


def _make_spmm_esplit(fc):
  @functools.partial(
      pl.kernel,
      out_type=jax.ShapeDtypeStruct((NSC * N_PAD, fc), jnp.float32),
      mesh=_sc_mesh(),
      scratch_types=[
          pltpu.VMEM((KU, B_H), jnp.int32),
          pltpu.VMEM((KU, B_H), jnp.int32),
          pltpu.VMEM((B_H, fc), jnp.float32),
          pltpu.VMEM_SHARED((N_PAD, fc), jnp.float32),
          pltpu.SemaphoreType.DMA,
      ],
  )
  def spmm(h_hbm, src_hbm, dst_hbm, out_hbm, sstage, dstage, rows, acc, sem):
    c = lax.axis_index("c")
    s = lax.axis_index("s")
    w = c * NT + s
    r0 = s * RPT

    _zero_rows(rows, 40, fc)
    for b in range(RPT // 40):
      pltpu.sync_copy(rows.at[pl.ds(0, 40)],
                      acc.at[pl.ds(r0 + b * 40, 40)])
    plsc.subcore_barrier()

    def rnd(g, carry):
      pltpu.sync_copy(src_hbm.at[w].at[pl.ds(g * KU, KU)], sstage)
      pltpu.sync_copy(dst_hbm.at[w].at[pl.ds(g * KU, KU)], dstage)
      for jj in range(KU):
        pltpu.async_copy(h_hbm.at[sstage.at[jj]], rows, sem).wait()
        pltpu.sync_copy(rows, acc.at[dstage.at[jj]], add=True)
      return carry
    lax.fori_loop(0, NB_H // KU, rnd, 0)
    plsc.subcore_barrier()
    pltpu.sync_copy(acc.at[pl.ds(r0, RPT)],
                    out_hbm.at[pl.ds(c * N_PAD + r0, RPT)])
    return

  return spmm


def _make_deg():
  @functools.partial(
      pl.kernel,
      out_type=jax.ShapeDtypeStruct((NSC * N_PAD, 128), jnp.float32),
      mesh=_sc_mesh(),
      scratch_types=[
          pltpu.VMEM((KU, B_H), jnp.int32),
          pltpu.VMEM((B_H, 128), jnp.float32),
          pltpu.VMEM_SHARED((N_PAD, 128), jnp.float32),
      ],
  )
  def deg(dst_hbm, out_hbm, dstage, ones_v, acc):
    c = lax.axis_index("c")
    s = lax.axis_index("s")
    w = c * NT + s
    r0 = s * RPT

    _zero_rows(ones_v, B_H, 128)
    for b in range(RPT // 40):
      pltpu.sync_copy(ones_v.at[pl.ds(0, 40)],
                      acc.at[pl.ds(r0 + b * 40, 40)])
    plsc.subcore_barrier()

    o16 = jnp.ones((16,), jnp.float32)

    def orow(i, carry):
      ones_v[i, pl.ds(0, 16)] = o16
      return carry
    lax.fori_loop(0, B_H, orow, 0)

    def rnd(g, carry):
      pltpu.sync_copy(dst_hbm.at[w].at[pl.ds(g * KU, KU)], dstage)
      for jj in range(KU):
        pltpu.sync_copy(ones_v, acc.at[dstage.at[jj]], add=True)
      return carry
    lax.fori_loop(0, NB_H // KU, rnd, 0)
    plsc.subcore_barrier()
    pltpu.sync_copy(acc.at[pl.ds(r0, RPT)],
                    out_hbm.at[pl.ds(c * N_PAD + r0, RPT)])
    return

  return deg


def _tc_dinv(degp_ref, dv_ref):
  dsum = jnp.sum(degp_ref[...], axis=(0, 2)) * (1.0 / 16.0)
  dv_ref[0, 0] = jnp.where(dsum > 0,
                           lax.rsqrt(jnp.where(dsum > 0, dsum, 1.0)), 0.0)


def _tc_a(x_ref, dv_ref, wi_ref, wr_ref, bv_ref, h0_ref, r1_ref):
  xb = x_ref[...]
  d = dv_ref[0, 0][:, None]
  h0_ref[0] = jnp.dot(xb, wi_ref[0], preferred_element_type=jnp.float32) * d
  r1_ref[0] = (jnp.dot(xb, wr_ref[0], preferred_element_type=jnp.float32)
               + bv_ref[0, 0][None, :])


def _tc_b(ag_ref, r1_ref, dv_ref, w_ref, h1_ref):
  d = dv_ref[0, 0][:, None]
  t = jnp.maximum(ag_ref[...] * d[None] + r1_ref[...], 0.0)
  a = jnp.concatenate([t[0], t[1]], axis=1)
  h1_ref[0] = jnp.dot(a, w_ref[0], preferred_element_type=jnp.float32) * d


def _tc_c(ag_ref, r1_ref, dv_ref, wi_ref, wr_ref, bv_ref, h2_ref, r2_ref):
  d = dv_ref[0, 0][:, None]
  o = jnp.maximum(ag_ref[...] * d[None] + r1_ref[...], 0.0)
  xi = 0.5 * (jnp.concatenate([o[0], o[1]], axis=1)
              + jnp.concatenate([o[2], o[3]], axis=1))
  h2_ref[...] = jnp.concatenate(
      [jnp.dot(xi, wi_ref[0], preferred_element_type=jnp.float32),
       jnp.dot(xi, wi_ref[1], preferred_element_type=jnp.float32)],
      axis=1) * d
  r2_ref[...] = (jnp.concatenate(
      [jnp.dot(xi, wr_ref[0], preferred_element_type=jnp.float32),
       jnp.dot(xi, wr_ref[1], preferred_element_type=jnp.float32)],
      axis=1) + bv_ref[0][None, :])


def _tc_d(agp_ref, r2_ref, dv_ref, w_ref, h_ref):
  d = dv_ref[0, 0][:, None]
  t = (agp_ref[0] + agp_ref[1]) * d + r2_ref[...]
  h_ref[...] = jnp.concatenate(
      [jnp.dot(t[:, :NCLS], w_ref[0], preferred_element_type=jnp.float32),
       jnp.dot(t[:, NCLS:], w_ref[1], preferred_element_type=jnp.float32)],
      axis=1) * d


def _tc_e(agp_ref, r2_ref, dv_ref, out_ref):
  d = dv_ref[0, 0][:, None]
  o = (agp_ref[0] + agp_ref[1]) * d + r2_ref[...]
  m = 0.5 * (o[:, :NCLS] + o[:, NCLS:])
  z = m - jnp.max(m, axis=1, keepdims=True)
  out_ref[...] = z - jnp.log(jnp.sum(jnp.exp(z), axis=1, keepdims=True))


_spmm_l1 = _make_spmm_fsplit(4, 2, 128)
_spmm_l2 = _make_spmm_esplit(128)
_deg_k = _make_deg()

_DV_SPEC2 = pl.BlockSpec((1, 1, RB), lambda i, c: (i, 0, 0))
_DV_SPEC1 = pl.BlockSpec((1, 1, RB), lambda i: (i, 0, 0))


def kernel(x, edge_index, init_w1, w1, root_w1, b1, init_w2, w2, root_w2, b2):
  f32 = jnp.float32
  npad = E_PAD - E
  pad_src = jnp.zeros((npad,), jnp.int32)
  pad_dst = N + (jnp.arange(npad, dtype=jnp.int32) % (N_PAD - N))
  srcp = jnp.concatenate([edge_index[0], pad_src])
  dstp = jnp.concatenate([edge_index[1], pad_dst])
  src3 = srcp.reshape(NT, NB_E, B_E)
  dst3 = dstp.reshape(NT, NB_E, B_E)
  src4 = srcp.reshape(NSC * NT, NB_H, B_H)
  dst4 = dstp.reshape(NSC * NT, NB_H, B_H)

  wi1 = init_w1
  wr1 = root_w1.reshape(K, F_IN, HID)
  b1v = b1.reshape(K, 1, HID)
  w1t = w1.reshape(K, HID, HID)
  wi2 = init_w2
  wr2 = root_w2.reshape(K, HID, NCLS)
  b2v = b2.reshape(1, K * NCLS)
  w2t = w2.reshape(K, NCLS, NCLS)

  degp = _deg_k(dst4).reshape(NSC, N_PAD, 128)
  dinv = pl.pallas_call(
      _tc_dinv,
      grid=(GR,),
      in_specs=[pl.BlockSpec((NSC, RB, 128), lambda i: (0, i, 0))],
      out_specs=pl.BlockSpec((1, 1, RB), lambda i: (i, 0, 0)),
      out_shape=jax.ShapeDtypeStruct((GR, 1, RB), f32),
  )(degp)

  h0, r1 = pl.pallas_call(
      _tc_a,
      grid=(GR, 4),
      in_specs=[
          pl.BlockSpec((RB, F_IN), lambda i, c: (i, 0)),
          _DV_SPEC2,
          pl.BlockSpec((1, F_IN, 128), lambda i, c: (c // 2, 0, c % 2)),
          pl.BlockSpec((1, F_IN, 128), lambda i, c: (c // 2, 0, c % 2)),
          pl.BlockSpec((1, 1, 128), lambda i, c: (c // 2, 0, c % 2)),
      ],
      out_specs=[
          pl.BlockSpec((1, RB, 128), lambda i, c: (c, i, 0)),
          pl.BlockSpec((1, RB, 128), lambda i, c: (c, i, 0)),
      ],
      out_shape=[
          jax.ShapeDtypeStruct((4, N, 128), f32),
          jax.ShapeDtypeStruct((4, N, 128), f32),
      ],
  )(x, dinv, wi1, wr1, b1v)

  ag0 = _spmm_l1(h0.reshape(4 * N, 128), src3, dst3).reshape(4, N_PAD, 128)

  h1 = pl.pallas_call(
      _tc_b,
      grid=(GR, 4),
      in_specs=[
          pl.BlockSpec((2, RB, 128), lambda i, c: (c // 2, i, 0)),
          pl.BlockSpec((2, RB, 128), lambda i, c: (c // 2, i, 0)),
          _DV_SPEC2,
          pl.BlockSpec((1, HID, 128), lambda i, c: (c // 2, 0, c % 2)),
      ],
      out_specs=pl.BlockSpec((1, RB, 128), lambda i, c: (c, i, 0)),
      out_shape=jax.ShapeDtypeStruct((4, N, 128), f32),
  )(ag0, r1, dinv, w1t)

  ag1 = _spmm_l1(h1.reshape(4 * N, 128), src3, dst3).reshape(4, N_PAD, 128)

  h2, r2 = pl.pallas_call(
      _tc_c,
      grid=(GR,),
      in_specs=[
          pl.BlockSpec((4, RB, 128), lambda i: (0, i, 0)),
          pl.BlockSpec((4, RB, 128), lambda i: (0, i, 0)),
          _DV_SPEC1,
          pl.BlockSpec((K, HID, NCLS), lambda i: (0, 0, 0)),
          pl.BlockSpec((K, HID, NCLS), lambda i: (0, 0, 0)),
          pl.BlockSpec((1, K * NCLS), lambda i: (0, 0)),
      ],
      out_specs=[
          pl.BlockSpec((RB, K * NCLS), lambda i: (i, 0)),
          pl.BlockSpec((RB, K * NCLS), lambda i: (i, 0)),
      ],
      out_shape=[
          jax.ShapeDtypeStruct((N, K * NCLS), f32),
          jax.ShapeDtypeStruct((N, K * NCLS), f32),
      ],
  )(ag1, r1, dinv, wi2, wr2, b2v)

  ag2 = _spmm_l2(h2, src4, dst4).reshape(NSC, N_PAD, 128)

  h3 = pl.pallas_call(
      _tc_d,
      grid=(GR,),
      in_specs=[
          pl.BlockSpec((NSC, RB, 128), lambda i: (0, i, 0)),
          pl.BlockSpec((RB, K * NCLS), lambda i: (i, 0)),
          _DV_SPEC1,
          pl.BlockSpec((K, NCLS, NCLS), lambda i: (0, 0, 0)),
      ],
      out_specs=pl.BlockSpec((RB, K * NCLS), lambda i: (i, 0)),
      out_shape=jax.ShapeDtypeStruct((N, K * NCLS), f32),
  )(ag2, r2, dinv, w2t)

  ag3 = _spmm_l2(h3, src4, dst4).reshape(NSC, N_PAD, 128)

  out = pl.pallas_call(
      _tc_e,
      grid=(GR,),
      in_specs=[
          pl.BlockSpec((NSC, RB, 128), lambda i: (0, i, 0)),
          pl.BlockSpec((RB, K * NCLS), lambda i: (i, 0)),
          _DV_SPEC1,
      ],
      out_specs=pl.BlockSpec((RB, NCLS), lambda i: (i, 0)),
      out_shape=jax.ShapeDtypeStruct((N, NCLS), f32),
  )(ag3, r2, dinv)

  return out


# speedup vs baseline: 29.8725x; 1.0970x over previous
import functools

import jax
import jax.numpy as jnp
from jax import lax
from jax.experimental import pallas as pl
from jax.experimental.pallas import tpu as pltpu
from jax.experimental.pallas import tpu_sc as plsc

N = 10000
E = 160000
F_IN = 256
HID = 256
NCLS = 64
K = 2

N_PAD = 10240
NT = 16
NSC = 2
RPT = N_PAD // NT
E_PAD = 163840
B_E = 128
NB_E = E_PAD // NT // B_E
B_H = 128
NB_H = E_PAD // (NSC * NT) // B_H

KU = 8
RB = 400
GR = N // RB


def _sc_mesh():
  return plsc.VectorSubcoreMesh(core_axis_name="c", subcore_axis_name="s")


def _zero_rows(rows, nrows, fc):
  z16 = jnp.zeros((16,), jnp.float32)

  def zrow(i, carry):
    def zcol(jj, carry2):
      rows[i, pl.ds(jj * 16, 16)] = z16
      return carry2
    return lax.fori_loop(0, fc // 16, zcol, carry)
  lax.fori_loop(0, nrows, zrow, 0)


def _make_spmm_fsplit(ch, cpc, fc):
  @functools.partial(
      pl.kernel,
      out_type=jax.ShapeDtypeStruct((ch * N_PAD, fc), jnp.float32),
      mesh=_sc_mesh(),
      scratch_types=[
          pltpu.VMEM((KU, B_E), jnp.int32),
          pltpu.VMEM((KU, B_E), jnp.int32),
          pltpu.VMEM((B_E, fc), jnp.float32),
          pltpu.VMEM((B_E, fc), jnp.float32),
          pltpu.VMEM_SHARED((N_PAD, fc), jnp.float32),
          pltpu.SemaphoreType.DMA,
          pltpu.SemaphoreType.DMA,
      ],
  )
  def spmm(h_hbm, src_hbm, dst_hbm, out_hbm, sstage, dstage, rows, rows2,
           acc, sem, sem2):
    c = lax.axis_index("c")
    s = lax.axis_index("s")
    r0 = s * RPT

    for j in range(cpc):
      cg = c * cpc + j
      base = cg * N

      _zero_rows(rows, 40, fc)
      for b in range(RPT // 40):
        pltpu.sync_copy(rows.at[pl.ds(0, 40)],
                        acc.at[pl.ds(r0 + b * 40, 40)])
      plsc.subcore_barrier()

      def rnd(g, carry):
        pltpu.sync_copy(src_hbm.at[s].at[pl.ds(g * KU, KU)], sstage)
        pltpu.sync_copy(dst_hbm.at[s].at[pl.ds(g * KU, KU)], dstage)
        for jj in range(KU):
          for v in range(B_E // 16):
            sl = pl.ds(v * 16, 16)
            sstage[jj, sl] = sstage[jj, sl] + base
        bufs = (rows, rows2)
        sems = (sem, sem2)
        pltpu.async_copy(h_hbm.at[sstage.at[0]], bufs[0], sems[0])
        for jj in range(KU):
          b = jj % 2
          pltpu.make_async_copy(h_hbm.at[sstage.at[jj]], bufs[b],
                                sems[b]).wait()
          if jj + 1 < KU:
            nb = (jj + 1) % 2
            pltpu.async_copy(h_hbm.at[sstage.at[jj + 1]], bufs[nb], sems[nb])
          pltpu.sync_copy(bufs[b], acc.at[dstage.at[jj]], add=True)
        return carry
      lax.fori_loop(0, NB_E // KU, rnd, 0)
      plsc.subcore_barrier()
      pltpu.sync_copy(acc.at[pl.ds(r0, RPT)],
                      out_hbm.at[pl.ds(cg * N_PAD + r0, RPT)])
    return

  return spmm


def _make_spmm_esplit(fc):
  @functools.partial(
      pl.kernel,
      out_type=jax.ShapeDtypeStruct((NSC * N_PAD, fc), jnp.float32),
      mesh=_sc_mesh(),
      scratch_types=[
          pltpu.VMEM((KU, B_H), jnp.int32),
          pltpu.VMEM((KU, B_H), jnp.int32),
          pltpu.VMEM((B_H, fc), jnp.float32),
          pltpu.VMEM((B_H, fc), jnp.float32),
          pltpu.VMEM_SHARED((N_PAD, fc), jnp.float32),
          pltpu.SemaphoreType.DMA,
          pltpu.SemaphoreType.DMA,
      ],
  )
  def spmm(h_hbm, src_hbm, dst_hbm, out_hbm, sstage, dstage, rows, rows2,
           acc, sem, sem2):
    c = lax.axis_index("c")
    s = lax.axis_index("s")
    w = c * NT + s
    r0 = s * RPT

    _zero_rows(rows, 40, fc)
    for b in range(RPT // 40):
      pltpu.sync_copy(rows.at[pl.ds(0, 40)],
                      acc.at[pl.ds(r0 + b * 40, 40)])
    plsc.subcore_barrier()

    def rnd(g, carry):
      pltpu.sync_copy(src_hbm.at[w].at[pl.ds(g * KU, KU)], sstage)
      pltpu.sync_copy(dst_hbm.at[w].at[pl.ds(g * KU, KU)], dstage)
      bufs = (rows, rows2)
      sems = (sem, sem2)
      pltpu.async_copy(h_hbm.at[sstage.at[0]], bufs[0], sems[0])
      for jj in range(KU):
        b = jj % 2
        pltpu.make_async_copy(h_hbm.at[sstage.at[jj]], bufs[b],
                              sems[b]).wait()
        if jj + 1 < KU:
          nb = (jj + 1) % 2
          pltpu.async_copy(h_hbm.at[sstage.at[jj + 1]], bufs[nb], sems[nb])
        pltpu.sync_copy(bufs[b], acc.at[dstage.at[jj]], add=True)
      return carry
    lax.fori_loop(0, NB_H // KU, rnd, 0)
    plsc.subcore_barrier()
    pltpu.sync_copy(acc.at[pl.ds(r0, RPT)],
                    out_hbm.at[pl.ds(c * N_PAD + r0, RPT)])
    return

  return spmm


def _make_deg():
  @functools.partial(
      pl.kernel,
      out_type=jax.ShapeDtypeStruct((NSC * N_PAD, 128), jnp.float32),
      mesh=_sc_mesh(),
      scratch_types=[
          pltpu.VMEM((KU, B_H), jnp.int32),
          pltpu.VMEM((B_H, 128), jnp.float32),
          pltpu.VMEM_SHARED((N_PAD, 128), jnp.float32),
      ],
  )
  def deg(dst_hbm, out_hbm, dstage, ones_v, acc):
    c = lax.axis_index("c")
    s = lax.axis_index("s")
    w = c * NT + s
    r0 = s * RPT

    _zero_rows(ones_v, B_H, 128)
    for b in range(RPT // 40):
      pltpu.sync_copy(ones_v.at[pl.ds(0, 40)],
                      acc.at[pl.ds(r0 + b * 40, 40)])
    plsc.subcore_barrier()

    o16 = jnp.ones((16,), jnp.float32)

    def orow(i, carry):
      ones_v[i, pl.ds(0, 16)] = o16
      return carry
    lax.fori_loop(0, B_H, orow, 0)

    def rnd(g, carry):
      pltpu.sync_copy(dst_hbm.at[w].at[pl.ds(g * KU, KU)], dstage)
      for jj in range(KU):
        pltpu.sync_copy(ones_v, acc.at[dstage.at[jj]], add=True)
      return carry
    lax.fori_loop(0, NB_H // KU, rnd, 0)
    plsc.subcore_barrier()
    pltpu.sync_copy(acc.at[pl.ds(r0, RPT)],
                    out_hbm.at[pl.ds(c * N_PAD + r0, RPT)])
    return

  return deg


def _tc_dinv(degp_ref, dv_ref):
  dsum = jnp.sum(degp_ref[...], axis=(0, 2)) * (1.0 / 16.0)
  dv_ref[0, 0] = jnp.where(dsum > 0,
                           lax.rsqrt(jnp.where(dsum > 0, dsum, 1.0)), 0.0)


def _tc_a(x_ref, dv_ref, wi_ref, wr_ref, bv_ref, h0_ref, r1_ref):
  xb = x_ref[...]
  d = dv_ref[0, 0][:, None]
  h0_ref[0] = jnp.dot(xb, wi_ref[0], preferred_element_type=jnp.float32) * d
  r1_ref[0] = (jnp.dot(xb, wr_ref[0], preferred_element_type=jnp.float32)
               + bv_ref[0, 0][None, :])


def _tc_b(ag_ref, r1_ref, dv_ref, w_ref, h1_ref):
  d = dv_ref[0, 0][:, None]
  t = jnp.maximum(ag_ref[...] * d[None] + r1_ref[...], 0.0)
  a = jnp.concatenate([t[0], t[1]], axis=1)
  h1_ref[0] = jnp.dot(a, w_ref[0], preferred_element_type=jnp.float32) * d


def _tc_c(ag_ref, r1_ref, dv_ref, wi_ref, wr_ref, bv_ref, h2_ref, r2_ref):
  d = dv_ref[0, 0][:, None]
  o = jnp.maximum(ag_ref[...] * d[None] + r1_ref[...], 0.0)
  xi = 0.5 * (jnp.concatenate([o[0], o[1]], axis=1)
              + jnp.concatenate([o[2], o[3]], axis=1))
  h2_ref[...] = jnp.concatenate(
      [jnp.dot(xi, wi_ref[0], preferred_element_type=jnp.float32),
       jnp.dot(xi, wi_ref[1], preferred_element_type=jnp.float32)],
      axis=1) * d
  r2_ref[...] = (jnp.concatenate(
      [jnp.dot(xi, wr_ref[0], preferred_element_type=jnp.float32),
       jnp.dot(xi, wr_ref[1], preferred_element_type=jnp.float32)],
      axis=1) + bv_ref[0][None, :])


def _tc_d(agp_ref, r2_ref, dv_ref, w_ref, h_ref):
  d = dv_ref[0, 0][:, None]
  t = (agp_ref[0] + agp_ref[1]) * d + r2_ref[...]
  h_ref[...] = jnp.concatenate(
      [jnp.dot(t[:, :NCLS], w_ref[0], preferred_element_type=jnp.float32),
       jnp.dot(t[:, NCLS:], w_ref[1], preferred_element_type=jnp.float32)],
      axis=1) * d


def _tc_e(agp_ref, r2_ref, dv_ref, out_ref):
  d = dv_ref[0, 0][:, None]
  o = (agp_ref[0] + agp_ref[1]) * d + r2_ref[...]
  m = 0.5 * (o[:, :NCLS] + o[:, NCLS:])
  z = m - jnp.max(m, axis=1, keepdims=True)
  out_ref[...] = z - jnp.log(jnp.sum(jnp.exp(z), axis=1, keepdims=True))


_spmm_l1 = _make_spmm_fsplit(4, 2, 128)
_spmm_l2 = _make_spmm_esplit(128)
_deg_k = _make_deg()

_DV_SPEC2 = pl.BlockSpec((1, 1, RB), lambda i, c: (i, 0, 0))
_DV_SPEC1 = pl.BlockSpec((1, 1, RB), lambda i: (i, 0, 0))


def kernel(x, edge_index, init_w1, w1, root_w1, b1, init_w2, w2, root_w2, b2):
  f32 = jnp.float32
  npad = E_PAD - E
  pad_src = jnp.zeros((npad,), jnp.int32)
  pad_dst = N + (jnp.arange(npad, dtype=jnp.int32) % (N_PAD - N))
  srcp = jnp.concatenate([edge_index[0], pad_src])
  dstp = jnp.concatenate([edge_index[1], pad_dst])
  src3 = srcp.reshape(NT, NB_E, B_E)
  dst3 = dstp.reshape(NT, NB_E, B_E)
  src4 = srcp.reshape(NSC * NT, NB_H, B_H)
  dst4 = dstp.reshape(NSC * NT, NB_H, B_H)

  wi1 = init_w1
  wr1 = root_w1.reshape(K, F_IN, HID)
  b1v = b1.reshape(K, 1, HID)
  w1t = w1.reshape(K, HID, HID)
  wi2 = init_w2
  wr2 = root_w2.reshape(K, HID, NCLS)
  b2v = b2.reshape(1, K * NCLS)
  w2t = w2.reshape(K, NCLS, NCLS)

  degp = _deg_k(dst4).reshape(NSC, N_PAD, 128)
  dinv = pl.pallas_call(
      _tc_dinv,
      grid=(GR,),
      in_specs=[pl.BlockSpec((NSC, RB, 128), lambda i: (0, i, 0))],
      out_specs=pl.BlockSpec((1, 1, RB), lambda i: (i, 0, 0)),
      out_shape=jax.ShapeDtypeStruct((GR, 1, RB), f32),
  )(degp)

  h0, r1 = pl.pallas_call(
      _tc_a,
      grid=(GR, 4),
      in_specs=[
          pl.BlockSpec((RB, F_IN), lambda i, c: (i, 0)),
          _DV_SPEC2,
          pl.BlockSpec((1, F_IN, 128), lambda i, c: (c // 2, 0, c % 2)),
          pl.BlockSpec((1, F_IN, 128), lambda i, c: (c // 2, 0, c % 2)),
          pl.BlockSpec((1, 1, 128), lambda i, c: (c // 2, 0, c % 2)),
      ],
      out_specs=[
          pl.BlockSpec((1, RB, 128), lambda i, c: (c, i, 0)),
          pl.BlockSpec((1, RB, 128), lambda i, c: (c, i, 0)),
      ],
      out_shape=[
          jax.ShapeDtypeStruct((4, N, 128), f32),
          jax.ShapeDtypeStruct((4, N, 128), f32),
      ],
  )(x, dinv, wi1, wr1, b1v)

  ag0 = _spmm_l1(h0.reshape(4 * N, 128), src3, dst3).reshape(4, N_PAD, 128)

  h1 = pl.pallas_call(
      _tc_b,
      grid=(GR, 4),
      in_specs=[
          pl.BlockSpec((2, RB, 128), lambda i, c: (c // 2, i, 0)),
          pl.BlockSpec((2, RB, 128), lambda i, c: (c // 2, i, 0)),
          _DV_SPEC2,
          pl.BlockSpec((1, HID, 128), lambda i, c: (c // 2, 0, c % 2)),
      ],
      out_specs=pl.BlockSpec((1, RB, 128), lambda i, c: (c, i, 0)),
      out_shape=jax.ShapeDtypeStruct((4, N, 128), f32),
  )(ag0, r1, dinv, w1t)

  ag1 = _spmm_l1(h1.reshape(4 * N, 128), src3, dst3).reshape(4, N_PAD, 128)

  h2, r2 = pl.pallas_call(
      _tc_c,
      grid=(GR,),
      in_specs=[
          pl.BlockSpec((4, RB, 128), lambda i: (0, i, 0)),
          pl.BlockSpec((4, RB, 128), lambda i: (0, i, 0)),
          _DV_SPEC1,
          pl.BlockSpec((K, HID, NCLS), lambda i: (0, 0, 0)),
          pl.BlockSpec((K, HID, NCLS), lambda i: (0, 0, 0)),
          pl.BlockSpec((1, K * NCLS), lambda i: (0, 0)),
      ],
      out_specs=[
          pl.BlockSpec((RB, K * NCLS), lambda i: (i, 0)),
          pl.BlockSpec((RB, K * NCLS), lambda i: (i, 0)),
      ],
      out_shape=[
          jax.ShapeDtypeStruct((N, K * NCLS), f32),
          jax.ShapeDtypeStruct((N, K * NCLS), f32),
      ],
  )(ag1, r1, dinv, wi2, wr2, b2v)

  ag2 = _spmm_l2(h2, src4, dst4).reshape(NSC, N_PAD, 128)

  h3 = pl.pallas_call(
      _tc_d,
      grid=(GR,),
      in_specs=[
          pl.BlockSpec((NSC, RB, 128), lambda i: (0, i, 0)),
          pl.BlockSpec((RB, K * NCLS), lambda i: (i, 0)),
          _DV_SPEC1,
          pl.BlockSpec((K, NCLS, NCLS), lambda i: (0, 0, 0)),
      ],
      out_specs=pl.BlockSpec((RB, K * NCLS), lambda i: (i, 0)),
      out_shape=jax.ShapeDtypeStruct((N, K * NCLS), f32),
  )(ag2, r2, dinv, w2t)

  ag3 = _spmm_l2(h3, src4, dst4).reshape(NSC, N_PAD, 128)

  out = pl.pallas_call(
      _tc_e,
      grid=(GR,),
      in_specs=[
          pl.BlockSpec((NSC, RB, 128), lambda i: (0, i, 0)),
          pl.BlockSpec((RB, K * NCLS), lambda i: (i, 0)),
          _DV_SPEC1,
      ],
      out_specs=pl.BlockSpec((RB, NCLS), lambda i: (i, 0)),
      out_shape=jax.ShapeDtypeStruct((N, NCLS), f32),
  )(ag3, r2, dinv)

  return out
